# baseline (speedup 1.0000x reference)
import functools
import math

import jax
import jax.numpy as jnp
from jax import lax
from jax.experimental import pallas as pl
from jax.experimental.pallas import tpu as pltpu
from jax.experimental.pallas import tpu_sc as plsc

_K_SEL = 8
_S_BLK1 = 4096
_S_BLK3 = 2048


def _gelu_exact(x):
    return 0.5 * x * (1.0 + lax.erf(x * (1.0 / math.sqrt(2.0))))


def _scores_body(idx_ref, acts_ref, w_ref, scores_ref, spw_ref, *, s_total):
    si = pl.program_id(1)
    n_in = w_ref.shape[1]
    io = lax.broadcasted_iota(jnp.int32, (n_in, idx_ref.shape[2]), 0)
    oh = (io == idx_ref[0]).astype(jnp.float32)
    spw = jax.lax.dot(w_ref[...], oh)

    @pl.when(si == 0)
    def _():
        spw_ref[0] = jnp.concatenate(
            [spw, jnp.zeros((spw.shape[0], 128 - spw.shape[1]), jnp.float32)],
            axis=1)

    x = acts_ref[0]
    nt = (((1,), (1,)), ((), ()))
    x_hi = x.astype(jnp.bfloat16)
    x_lo = (x - x_hi.astype(jnp.float32)).astype(jnp.bfloat16)
    w_hi = spw.astype(jnp.bfloat16)
    w_lo = (spw - w_hi.astype(jnp.float32)).astype(jnp.bfloat16)
    f32 = jnp.float32
    pa = (lax.dot_general(x_hi, w_hi, nt, preferred_element_type=f32)
          + (lax.dot_general(x_hi, w_lo, nt, preferred_element_type=f32)
             + lax.dot_general(x_lo, w_hi, nt, preferred_element_type=f32)))
    u = pa * lax.erf(pa * (1.0 / math.sqrt(2.0)))
    ones = jnp.full((1, x.shape[0]), 0.5 / s_total, jnp.float32)
    colsum = jax.lax.dot(ones, x)
    part = (jax.lax.dot(ones, u)
            + lax.dot_general(colsum, spw, nt))

    @pl.when(si == 0)
    def _():
        scores_ref[0] = part

    @pl.when(si != 0)
    def _():
        scores_ref[0] = scores_ref[0] + part


def _route_body(scores_ref, spw_ref, po_ref, spw_sel_ref, po_sel_ref):
    s = scores_ref[...]
    bq, n_proc = s.shape
    iota_p = lax.broadcasted_iota(jnp.int32, (bq, n_proc), 1)
    rows = []
    for _ in range(_K_SEL):
        m = jnp.max(s, axis=1, keepdims=True)
        idx = jnp.min(jnp.where(s == m, iota_p, n_proc), axis=1, keepdims=True)
        hit = iota_p == idx
        rows.append(hit.astype(jnp.float32))
        s = jnp.where(hit, -jnp.inf, s)
    oh = jnp.stack(rows, axis=1)
    ps = jax.lax.dot(oh.reshape(bq * _K_SEL, n_proc), po_ref[...])
    po_sel_ref[...] = ps.reshape(bq, _K_SEL, po_ref.shape[1])
    for b in range(bq):
        spw_sel_ref[b] = jax.lax.dot(oh[b], spw_ref[b])


def _route_sc_body(scores_hbm, spw2_hbm, po_hbm, wsel_hbm, posel_hbm,
                   scores_v, idxpo_v, idxspw_v, rows_po, rows_w, sem):
    wid = lax.axis_index("s") * 2 + lax.axis_index("c")
    n_proc = scores_hbm.shape[1]

    @pl.when(wid < scores_hbm.shape[0])
    def _():
        b = wid
        pltpu.sync_copy(scores_hbm.at[b], scores_v)
        iota = lax.broadcasted_iota(jnp.int32, (16,), 0)
        run_k = scores_v[pl.ds(0, 16)]
        run_i = iota
        for j in range(1, n_proc // 16):
            ck = scores_v[pl.ds(j * 16, 16)]
            ci = iota + j * 16
            rk, ri = lax.sort((run_k, run_i), num_keys=1)
            ck, ci = lax.sort((ck, ci), num_keys=1)
            ck, ci = lax.rev(ck, (0,)), lax.rev(ci, (0,))
            take = ck > rk
            run_k = jnp.where(take, ck, rk)
            run_i = jnp.where(take, ci, ri)
        _, run_i = lax.sort((run_k, run_i), num_keys=1)
        idxpo_v[...] = run_i
        pltpu.async_copy(po_hbm.at[idxpo_v], rows_po, sem).wait()
        pltpu.sync_copy(rows_po.at[pl.ds(_K_SEL, _K_SEL)], posel_hbm.at[b])
        idxspw_v[...] = run_i + b * n_proc
        pltpu.async_copy(spw2_hbm.at[idxspw_v], rows_w, sem).wait()
        pltpu.sync_copy(rows_w.at[pl.ds(_K_SEL, _K_SEL)], wsel_hbm.at[b])


def _out_body(acts_ref, wt_ref, po_sel_ref, out_ref):
    x = acts_ref[0]
    w = wt_ref[0][:, :x.shape[1]]
    a = _gelu_exact(lax.dot_general(x, w, (((1,), (1,)), ((), ()))))
    out_ref[0] = jax.lax.dot(a, po_sel_ref[0])


def kernel(selected_input_acts, input_idx, k_process, process_weights, process_outputs):
    del k_process
    B, S, k_in = selected_input_acts.shape
    n_proc, n_in = process_weights.shape
    d_model = process_outputs.shape[1]

    idx3 = input_idx.reshape(B, 1, k_in)

    scores, spw = pl.pallas_call(
        functools.partial(_scores_body, s_total=S),
        grid=(B, S // _S_BLK1),
        in_specs=[
            pl.BlockSpec((1, 1, k_in), lambda b, s: (b, 0, 0)),
            pl.BlockSpec((1, _S_BLK1, k_in), lambda b, s: (b, s, 0)),
            pl.BlockSpec((n_proc, n_in), lambda b, s: (0, 0)),
        ],
        out_specs=[
            pl.BlockSpec((1, 1, n_proc), lambda b, s: (b, 0, 0)),
            pl.BlockSpec((1, n_proc, 128), lambda b, s: (b, 0, 0)),
        ],
        out_shape=[
            jax.ShapeDtypeStruct((B, 1, n_proc), jnp.float32),
            jax.ShapeDtypeStruct((B, n_proc, 128), jnp.float32),
        ],
    )(idx3, selected_input_acts, process_weights)

    route = pl.kernel(
        _route_sc_body,
        out_type=[
            jax.ShapeDtypeStruct((B, _K_SEL, 128), jnp.float32),
            jax.ShapeDtypeStruct((B, _K_SEL, d_model), jnp.float32),
        ],
        mesh=plsc.VectorSubcoreMesh(
            core_axis_name="c", subcore_axis_name="s",
            num_cores=2, num_subcores=16),
        scratch_types=[
            pltpu.VMEM((n_proc,), jnp.float32),
            pltpu.VMEM((16,), jnp.int32),
            pltpu.VMEM((16,), jnp.int32),
            pltpu.VMEM((16, d_model), jnp.float32),
            pltpu.VMEM((16, 128), jnp.float32),
            pltpu.SemaphoreType.DMA,
        ],
        compiler_params=pltpu.CompilerParams(needs_layout_passes=False),
    )
    wt_sel, po_sel = route(
        scores.reshape(B, n_proc),
        spw.reshape(B * n_proc, 128),
        process_outputs,
    )

    out = pl.pallas_call(
        _out_body,
        grid=(B, S // _S_BLK3),
        in_specs=[
            pl.BlockSpec((1, _S_BLK3, k_in), lambda b, s: (b, s, 0)),
            pl.BlockSpec((1, _K_SEL, 128), lambda b, s: (b, 0, 0)),
            pl.BlockSpec((1, _K_SEL, d_model), lambda b, s: (b, 0, 0)),
        ],
        out_specs=pl.BlockSpec((1, _S_BLK3, d_model), lambda b, s: (b, s, 0)),
        out_shape=jax.ShapeDtypeStruct((B, S, d_model), jnp.float32),
    )(selected_input_acts, wt_sel, po_sel)
    return out


# speedup vs baseline: 1.2814x; 1.2814x over previous
import functools
import math

import jax
import jax.numpy as jnp
from jax import lax
from jax.experimental import pallas as pl
from jax.experimental.pallas import tpu as pltpu
from jax.experimental.pallas import tpu_sc as plsc

_K_SEL = 8
_S_BLK1 = 4096
_S_BLK3 = 2048


def _gelu_exact(x):
    return 0.5 * x * (1.0 + lax.erf(x * (1.0 / math.sqrt(2.0))))


def _scores_body(idx_ref, acts_ref, w_ref, scores_ref, spw_ref, *, s_total):
    si = pl.program_id(1)
    n_in = w_ref.shape[1]
    io = lax.broadcasted_iota(jnp.int32, (n_in, idx_ref.shape[2]), 0)
    oh = (io == idx_ref[0]).astype(jnp.float32)
    spw = jax.lax.dot(w_ref[...], oh)

    @pl.when(si == 0)
    def _():
        spw_ref[0] = jnp.concatenate(
            [spw, jnp.zeros((spw.shape[0], 128 - spw.shape[1]), jnp.float32)],
            axis=1)

    x = acts_ref[0]
    pa = lax.dot_general(x, spw, (((1,), (1,)), ((), ())))
    u = pa * lax.erf(pa * (1.0 / math.sqrt(2.0)))
    ones = jnp.full((1, x.shape[0]), 0.5 / s_total, jnp.float32)
    colsum = jax.lax.dot(ones, x)
    part = (jax.lax.dot(ones, u)
            + lax.dot_general(colsum, spw, (((1,), (1,)), ((), ()))))

    @pl.when(si == 0)
    def _():
        scores_ref[0] = part

    @pl.when(si != 0)
    def _():
        scores_ref[0] = scores_ref[0] + part


def _route_body(scores_ref, spw_ref, po_ref, spw_sel_ref, po_sel_ref):
    s = scores_ref[...]
    bq, n_proc = s.shape
    iota_p = lax.broadcasted_iota(jnp.int32, (bq, n_proc), 1)
    rows = []
    for _ in range(_K_SEL):
        m = jnp.max(s, axis=1, keepdims=True)
        idx = jnp.min(jnp.where(s == m, iota_p, n_proc), axis=1, keepdims=True)
        hit = iota_p == idx
        rows.append(hit.astype(jnp.float32))
        s = jnp.where(hit, -jnp.inf, s)
    oh = jnp.stack(rows, axis=1)
    ps = jax.lax.dot(oh.reshape(bq * _K_SEL, n_proc), po_ref[...])
    po_sel_ref[...] = ps.reshape(bq, _K_SEL, po_ref.shape[1])
    for b in range(bq):
        spw_sel_ref[b] = jax.lax.dot(oh[b], spw_ref[b])


def _route_sc_body(scores_hbm, spw2_hbm, po_hbm, wsel_hbm, posel_hbm,
                   scores_v, idxpo_v, idxspw_v, rows_po, rows_w, sem):
    wid = lax.axis_index("s") * 2 + lax.axis_index("c")
    n_proc = scores_hbm.shape[1]

    @pl.when(wid < scores_hbm.shape[0])
    def _():
        b = wid
        pltpu.sync_copy(scores_hbm.at[b], scores_v)
        iota = lax.broadcasted_iota(jnp.int32, (16,), 0)
        run_k = scores_v[pl.ds(0, 16)]
        run_i = iota
        for j in range(1, n_proc // 16):
            ck = scores_v[pl.ds(j * 16, 16)]
            ci = iota + j * 16
            rk, ri = lax.sort((run_k, run_i), num_keys=1)
            ck, ci = lax.sort((ck, ci), num_keys=1)
            ck, ci = lax.rev(ck, (0,)), lax.rev(ci, (0,))
            take = ck > rk
            run_k = jnp.where(take, ck, rk)
            run_i = jnp.where(take, ci, ri)
        _, run_i = lax.sort((run_k, run_i), num_keys=1)
        idxpo_v[...] = run_i
        pltpu.async_copy(po_hbm.at[idxpo_v], rows_po, sem).wait()
        pltpu.sync_copy(rows_po.at[pl.ds(_K_SEL, _K_SEL)], posel_hbm.at[b])
        idxspw_v[...] = run_i + b * n_proc
        pltpu.async_copy(spw2_hbm.at[idxspw_v], rows_w, sem).wait()
        pltpu.sync_copy(rows_w.at[pl.ds(_K_SEL, _K_SEL)], wsel_hbm.at[b])


def _out_body(acts_ref, wt_ref, po_sel_ref, out_ref):
    x = acts_ref[0]
    w = wt_ref[0][:, :x.shape[1]]
    a = _gelu_exact(lax.dot_general(x, w, (((1,), (1,)), ((), ()))))
    out_ref[0] = jax.lax.dot(a, po_sel_ref[0])


def kernel(selected_input_acts, input_idx, k_process, process_weights, process_outputs):
    del k_process
    B, S, k_in = selected_input_acts.shape
    n_proc, n_in = process_weights.shape
    d_model = process_outputs.shape[1]

    idx3 = input_idx.reshape(B, 1, k_in)

    scores, spw = pl.pallas_call(
        functools.partial(_scores_body, s_total=S),
        grid=(B, S // _S_BLK1),
        in_specs=[
            pl.BlockSpec((1, 1, k_in), lambda b, s: (b, 0, 0)),
            pl.BlockSpec((1, _S_BLK1, k_in), lambda b, s: (b, s, 0)),
            pl.BlockSpec((n_proc, n_in), lambda b, s: (0, 0)),
        ],
        out_specs=[
            pl.BlockSpec((1, 1, n_proc), lambda b, s: (b, 0, 0)),
            pl.BlockSpec((1, n_proc, 128), lambda b, s: (b, 0, 0)),
        ],
        out_shape=[
            jax.ShapeDtypeStruct((B, 1, n_proc), jnp.float32),
            jax.ShapeDtypeStruct((B, n_proc, 128), jnp.float32),
        ],
    )(idx3, selected_input_acts, process_weights)

    route = pl.kernel(
        _route_sc_body,
        out_type=[
            jax.ShapeDtypeStruct((B, _K_SEL, 128), jnp.float32),
            jax.ShapeDtypeStruct((B, _K_SEL, d_model), jnp.float32),
        ],
        mesh=plsc.VectorSubcoreMesh(
            core_axis_name="c", subcore_axis_name="s",
            num_cores=2, num_subcores=16),
        scratch_types=[
            pltpu.VMEM((n_proc,), jnp.float32),
            pltpu.VMEM((16,), jnp.int32),
            pltpu.VMEM((16,), jnp.int32),
            pltpu.VMEM((16, d_model), jnp.float32),
            pltpu.VMEM((16, 128), jnp.float32),
            pltpu.SemaphoreType.DMA,
        ],
        compiler_params=pltpu.CompilerParams(needs_layout_passes=False),
    )
    wt_sel, po_sel = route(
        scores.reshape(B, n_proc),
        spw.reshape(B * n_proc, 128),
        process_outputs,
    )

    out = pl.pallas_call(
        _out_body,
        grid=(B, S // _S_BLK3),
        in_specs=[
            pl.BlockSpec((1, _S_BLK3, k_in), lambda b, s: (b, s, 0)),
            pl.BlockSpec((1, _K_SEL, 128), lambda b, s: (b, 0, 0)),
            pl.BlockSpec((1, _K_SEL, d_model), lambda b, s: (b, 0, 0)),
        ],
        out_specs=pl.BlockSpec((1, _S_BLK3, d_model), lambda b, s: (b, s, 0)),
        out_shape=jax.ShapeDtypeStruct((B, S, d_model), jnp.float32),
    )(selected_input_acts, wt_sel, po_sel)
    return out
